# baseline (speedup 1.0000x reference)
import functools

import jax
import jax.numpy as jnp
from jax import lax
from jax.experimental import pallas as pl
from jax.experimental.pallas import tpu as pltpu
from jax.experimental.pallas import tpu_sc as plsc

_RC = 3.0
_N_NODES = 100000
_N_EDGES = 6400000
_E0_HALF = 2.0 * ((1.0 / _RC) ** 12 - (1.0 / _RC) ** 6)

_NC = 2
_NS = 16
_L = 16
_NW = _NC * _NS
_EPT = _N_EDGES // _NW
_CHUNK = 4000
_NCHUNK = _EPT // _CHUNK
_PAD = 100352
_SL = _PAD // _NS
_SLH = _SL // 2

_mesh = plsc.VectorSubcoreMesh(core_axis_name="c", subcore_axis_name="s")


@functools.partial(
    pl.kernel,
    mesh=_mesh,
    out_type=jax.ShapeDtypeStruct((_NW, _PAD), jnp.float32),
    scratch_types=[
        pltpu.VMEM((_CHUNK,), jnp.float32),
        pltpu.VMEM((_CHUNK,), jnp.float32),
        pltpu.VMEM((_CHUNK,), jnp.int32),
        pltpu.VMEM((_CHUNK,), jnp.int32),
        pltpu.VMEM((_PAD,), jnp.float32),
        pltpu.SemaphoreType.DMA,
        pltpu.SemaphoreType.DMA,
        pltpu.SemaphoreType.DMA,
        pltpu.SemaphoreType.DMA,
    ],
    compiler_params=pltpu.CompilerParams(needs_layout_passes=False),
)
def _sc_lj_scatter(
    dist_hbm, idx_hbm, out_hbm,
    dist0, dist1, idx0, idx1, acc_v,
    sd0, sd1, si0, si1,
):
    cid = lax.axis_index("c")
    sid = lax.axis_index("s")
    wid = sid * _NC + cid
    base = wid * _EPT

    def issue(off, db, ib, sdm, sim):
        pltpu.async_copy(dist_hbm.at[pl.ds(off, _CHUNK)], db, sdm)
        pltpu.async_copy(idx_hbm.at[pl.ds(off, _CHUNK)], ib, sim)

    def drain(db, ib, sdm, sim):
        pltpu.make_async_copy(dist_hbm.at[pl.ds(0, _CHUNK)], db, sdm).wait()
        pltpu.make_async_copy(idx_hbm.at[pl.ds(0, _CHUNK)], ib, sim).wait()

    def compute(db, ib):
        @plsc.parallel_loop(0, _CHUNK // _L, unroll=25)
        def _body(j):
            d = db[pl.ds(j * _L, _L)]
            ix = ib[pl.ds(j * _L, _L)]
            r = 1.0 / d
            r2 = r * r
            r6 = r2 * r2 * r2
            en = (r6 * r6 - r6) * 2.0 - _E0_HALF
            plsc.addupdate_scatter(acc_v, [ix], en)

    issue(base, dist0, idx0, sd0, si0)

    zero = jnp.zeros((_L,), jnp.float32)

    @plsc.parallel_loop(0, _PAD // (_L * 8))
    def _zero(i):
        for u in range(8):
            acc_v[pl.ds(i * (_L * 8) + u * _L, _L)] = zero

    def _outer(g, carry):
        off0 = base + (2 * g) * _CHUNK
        drain(dist0, idx0, sd0, si0)
        issue(off0 + _CHUNK, dist1, idx1, sd1, si1)
        compute(dist0, idx0)

        @pl.when(g < _NCHUNK // 2 - 1)
        def _():
            issue(off0 + 2 * _CHUNK, dist0, idx0, sd0, si0)

        drain(dist1, idx1, sd1, si1)
        compute(dist1, idx1)
        return carry

    lax.fori_loop(0, _NCHUNK // 2, _outer, 0)

    pltpu.sync_copy(acc_v, out_hbm.at[wid])


def _tc_reduce_body(p_ref, o_ref):
    o_ref[...] = jnp.sum(p_ref[...], axis=0)


_tc_reduce = pl.pallas_call(
    _tc_reduce_body,
    out_shape=jax.ShapeDtypeStruct((_PAD,), jnp.float32),
)


def kernel(dist, ind_1, ind_2):
    del ind_1
    idx = ind_2[:, 0].astype(jnp.int32)
    partials = _sc_lj_scatter(dist, idx)
    en = _tc_reduce(partials)
    return en[:_N_NODES]


# speedup vs baseline: 1.0547x; 1.0547x over previous
import functools

import jax
import jax.numpy as jnp
from jax import lax
from jax.experimental import pallas as pl
from jax.experimental.pallas import tpu as pltpu
from jax.experimental.pallas import tpu_sc as plsc

_RC = 3.0
_N_NODES = 100000
_N_EDGES = 6400000
_E0_HALF = 2.0 * ((1.0 / _RC) ** 12 - (1.0 / _RC) ** 6)

_NC = 2
_NS = 16
_L = 16
_NW = _NC * _NS
_EPT = _N_EDGES // _NW
_CHUNK = 4000
_NCHUNK = _EPT // _CHUNK
_PAD = 100352
_SL = _PAD // _NS
_SLH = _SL // 2

_mesh = plsc.VectorSubcoreMesh(core_axis_name="c", subcore_axis_name="s")


@functools.partial(
    pl.kernel,
    mesh=_mesh,
    out_type=jax.ShapeDtypeStruct((_NW, _PAD), jnp.float32),
    scratch_types=[
        pltpu.VMEM((_CHUNK,), jnp.float32),
        pltpu.VMEM((_CHUNK,), jnp.float32),
        pltpu.VMEM((_CHUNK,), jnp.int32),
        pltpu.VMEM((_CHUNK,), jnp.int32),
        pltpu.VMEM((_PAD,), jnp.float32),
        pltpu.SemaphoreType.DMA,
        pltpu.SemaphoreType.DMA,
        pltpu.SemaphoreType.DMA,
        pltpu.SemaphoreType.DMA,
    ],
    compiler_params=pltpu.CompilerParams(needs_layout_passes=False),
)
def _sc_lj_scatter(
    dist_hbm, idx_hbm, out_hbm,
    dist0, dist1, idx0, idx1, acc_v,
    sd0, sd1, si0, si1,
):
    cid = lax.axis_index("c")
    sid = lax.axis_index("s")
    wid = sid * _NC + cid
    base = wid * _EPT

    def issue(off, db, ib, sdm, sim):
        pltpu.async_copy(dist_hbm.at[pl.ds(off, _CHUNK)], db, sdm)
        pltpu.async_copy(idx_hbm.at[pl.ds(off, _CHUNK)], ib, sim)

    def drain(db, ib, sdm, sim):
        pltpu.make_async_copy(dist_hbm.at[pl.ds(0, _CHUNK)], db, sdm).wait()
        pltpu.make_async_copy(idx_hbm.at[pl.ds(0, _CHUNK)], ib, sim).wait()

    def compute(db, ib):
        @plsc.parallel_loop(0, _CHUNK // _L, unroll=10)
        def _body(j):
            d = db[pl.ds(j * _L, _L)]
            ix = ib[pl.ds(j * _L, _L)]
            r = 1.0 / d
            r2 = r * r
            r6 = r2 * r2 * r2
            en = (r6 * r6 - r6) * 2.0 - _E0_HALF
            plsc.addupdate_scatter(acc_v, [ix], en)

    issue(base, dist0, idx0, sd0, si0)

    zero = jnp.zeros((_L,), jnp.float32)

    @plsc.parallel_loop(0, _PAD // (_L * 8))
    def _zero(i):
        for u in range(8):
            acc_v[pl.ds(i * (_L * 8) + u * _L, _L)] = zero

    def _outer(g, carry):
        off0 = base + (2 * g) * _CHUNK
        drain(dist0, idx0, sd0, si0)
        issue(off0 + _CHUNK, dist1, idx1, sd1, si1)
        compute(dist0, idx0)

        @pl.when(g < _NCHUNK // 2 - 1)
        def _():
            issue(off0 + 2 * _CHUNK, dist0, idx0, sd0, si0)

        drain(dist1, idx1, sd1, si1)
        compute(dist1, idx1)
        return carry

    lax.fori_loop(0, _NCHUNK // 2, _outer, 0)

    pltpu.sync_copy(acc_v, out_hbm.at[wid])


def _tc_reduce_body(p_ref, o_ref):
    o_ref[...] = jnp.sum(p_ref[...], axis=0)


_tc_reduce = pl.pallas_call(
    _tc_reduce_body,
    out_shape=jax.ShapeDtypeStruct((_PAD,), jnp.float32),
)


def kernel(dist, ind_1, ind_2):
    del ind_1
    idx = ind_2[:, 0].astype(jnp.int32)
    partials = _sc_lj_scatter(dist, idx)
    en = _tc_reduce(partials)
    return en[:_N_NODES]
